# baseline (speedup 1.0000x reference)
import functools

import jax
import jax.numpy as jnp
from jax.experimental import pallas as pl
from jax.experimental.pallas import tpu as pltpu


def _lstm_body(x_ref, wih0_ref, whh0_ref, b0_ref, wih1_ref, whh1_ref, b1_ref,
               wfc_ref, bfc_ref, out_ref, gx_ref, seq_ref, xtm_ref,
               wih0s_ref, whh0s_ref, wih1s_ref, whh1s_ref, b0s_ref, b1s_ref,
               *, T, Bb, H):
    bf = jnp.bfloat16

    @pl.when(pl.program_id(0) == 0)
    def _prep():
        col = jax.lax.broadcasted_iota(jnp.int32, (1, 4 * H), 1)
        scl = jnp.where((col >= 2 * H) & (col < 3 * H), 1.0, 0.5)
        wih0s_ref[...] = (wih0_ref[...] * scl).astype(bf)
        whh0s_ref[...] = (whh0_ref[...] * scl).astype(bf)
        wih1s_ref[...] = (wih1_ref[...] * scl).astype(bf)
        whh1s_ref[...] = (whh1_ref[...] * scl).astype(bf)
        b0s_ref[...] = b0_ref[...] * scl
        b1s_ref[...] = b1_ref[...] * scl

    xtm_ref[...] = jnp.swapaxes(x_ref[...], 0, 1)
    wih0 = wih0s_ref[...]
    b0 = b0s_ref[...]
    for t in range(T):
        xt = xtm_ref[t].astype(bf)
        gx_ref[t] = (
            jnp.dot(xt, wih0, preferred_element_type=jnp.float32) + b0
        )

    def cell(pre, c):
        tt = jnp.tanh(pre)
        i_g = 0.5 + 0.5 * tt[:, 0 * H:1 * H]
        f_g = 0.5 + 0.5 * tt[:, 1 * H:2 * H]
        g_g = tt[:, 2 * H:3 * H]
        o_g = 0.5 + 0.5 * tt[:, 3 * H:4 * H]
        c = f_g * c + i_g * g_g
        h = o_g * jnp.tanh(c)
        return h, c

    whh0 = whh0s_ref[...]
    h = jnp.zeros((Bb, H), jnp.float32)
    c = h
    for t in range(T):
        pre = gx_ref[t] + jnp.dot(
            h.astype(bf), whh0, preferred_element_type=jnp.float32
        )
        h, c = cell(pre, c)
        seq_ref[t] = h.astype(bf)

    gx_ref[...] = (
        jnp.dot(seq_ref[...].reshape(T * Bb, H), wih1s_ref[...],
                preferred_element_type=jnp.float32).reshape(T, Bb, 4 * H)
        + b1s_ref[...]
    )

    whh1 = whh1s_ref[...]
    h = jnp.zeros((Bb, H), jnp.float32)
    c = h
    for t in range(T):
        pre = gx_ref[t] + jnp.dot(
            h.astype(bf), whh1, preferred_element_type=jnp.float32
        )
        h, c = cell(pre, c)

    out_ref[...] = (
        jnp.dot(h.astype(bf), wfc_ref[...].astype(bf),
                preferred_element_type=jnp.float32)
        + bfc_ref[...]
    )


@functools.partial(jax.jit, static_argnames=("block_b",))
def _forward(x, w_ih_0, w_hh_0, b_0, w_ih_1, w_hh_1, b_1, w_fc, b_fc,
             block_b=128):
    B, T, I = x.shape
    H = w_hh_0.shape[0]
    C = w_fc.shape[1]
    Bb = min(block_b, ((B + 7) // 8) * 8)
    Bp = ((B + Bb - 1) // Bb) * Bb
    if Bp != B:
        x = jnp.pad(x, ((0, Bp - B), (0, 0), (0, 0)))

    body = functools.partial(_lstm_body, T=T, Bb=Bb, H=H)
    bcast = lambda shape: pl.BlockSpec(shape, lambda i: (0,) * len(shape))
    out = pl.pallas_call(
        body,
        out_shape=jax.ShapeDtypeStruct((Bp, C), jnp.float32),
        grid=(Bp // Bb,),
        in_specs=[
            pl.BlockSpec((Bb, T, I), lambda i: (i, 0, 0)),
            bcast((I, 4 * H)), bcast((H, 4 * H)), bcast((1, 4 * H)),
            bcast((H, 4 * H)), bcast((H, 4 * H)), bcast((1, 4 * H)),
            bcast((H, C)), bcast((1, C)),
        ],
        out_specs=pl.BlockSpec((Bb, C), lambda i: (i, 0)),
        scratch_shapes=[
            pltpu.VMEM((T, Bb, 4 * H), jnp.float32),
            pltpu.VMEM((T, Bb, H), jnp.bfloat16),
            pltpu.VMEM((T, Bb, I), jnp.float32),
            pltpu.VMEM((I, 4 * H), jnp.bfloat16),
            pltpu.VMEM((H, 4 * H), jnp.bfloat16),
            pltpu.VMEM((H, 4 * H), jnp.bfloat16),
            pltpu.VMEM((H, 4 * H), jnp.bfloat16),
            pltpu.VMEM((1, 4 * H), jnp.float32),
            pltpu.VMEM((1, 4 * H), jnp.float32),
        ],
        compiler_params=pltpu.CompilerParams(
            dimension_semantics=("arbitrary",),
        ),
    )(x, w_ih_0, w_hh_0, b_0, w_ih_1, w_hh_1, b_1, w_fc, b_fc)
    return out[:B]


def kernel(x, w_ih_0, w_hh_0, b_0, w_ih_1, w_hh_1, b_1, w_fc, b_fc):
    return _forward(x, w_ih_0, w_hh_0, b_0, w_ih_1, w_hh_1, b_1, w_fc, b_fc)


# speedup vs baseline: 1.4656x; 1.4656x over previous
import functools

import jax
import jax.numpy as jnp
from jax.experimental import pallas as pl
from jax.experimental.pallas import tpu as pltpu

_CH = 128
_DEPTH = 2


def _lstm_body(x_hbm, wih0_ref, whh0_ref, b0_ref, wih1_ref, whh1_ref, b1_ref,
               wfc_ref, bfc_ref, out_ref, gx_ref, seq_ref, xtm_ref,
               xchunk_ref, sem, *, T, B, H):
    bf = jnp.bfloat16
    nch = B // _CH
    col = jax.lax.broadcasted_iota(jnp.int32, (1, 4 * H), 1)
    scl = jnp.where((col >= 2 * H) & (col < 3 * H), 1.0, 0.5)

    def chunk_copy(ch):
        return pltpu.make_async_copy(
            x_hbm.at[pl.ds(ch * _CH, _CH)], xchunk_ref.at[ch % _DEPTH],
            sem.at[ch % _DEPTH],
        )

    for ch in range(min(_DEPTH, nch)):
        chunk_copy(ch).start()

    wih0 = (wih0_ref[...] * scl).astype(bf)
    b0 = b0_ref[...] * scl

    for ch in range(nch):
        slot = ch % _DEPTH
        chunk_copy(ch).wait()
        xt = jnp.swapaxes(xchunk_ref[slot], 0, 1).astype(bf)
        xtm_ref[:, pl.ds(ch * _CH, _CH), :] = xt
        gx = jnp.dot(xt.reshape(T * _CH, x_hbm.shape[-1]), wih0,
                     preferred_element_type=jnp.float32) + b0
        gx_ref[:, pl.ds(ch * _CH, _CH), :] = gx.reshape(T, _CH, 4 * H)
        if ch + _DEPTH < nch:
            chunk_copy(ch + _DEPTH).start()

    def cell(pre, c):
        tt = jnp.tanh(pre)
        i_g = 0.5 + 0.5 * tt[:, 0 * H:1 * H]
        f_g = 0.5 + 0.5 * tt[:, 1 * H:2 * H]
        g_g = tt[:, 2 * H:3 * H]
        o_g = 0.5 + 0.5 * tt[:, 3 * H:4 * H]
        c = f_g * c + i_g * g_g
        h = o_g * jnp.tanh(c)
        return h, c

    whh0 = (whh0_ref[...] * scl).astype(bf)
    h = jnp.zeros((B, H), jnp.float32)
    c = h
    for t in range(T):
        pre = gx_ref[t] + jnp.dot(
            h.astype(bf), whh0, preferred_element_type=jnp.float32
        )
        h, c = cell(pre, c)
        seq_ref[t] = h.astype(bf)

    wih1 = (wih1_ref[...] * scl).astype(bf)
    gx_ref[...] = (
        jnp.dot(seq_ref[...].reshape(T * B, H), wih1,
                preferred_element_type=jnp.float32).reshape(T, B, 4 * H)
        + b1_ref[...] * scl
    )

    whh1 = (whh1_ref[...] * scl).astype(bf)
    h = jnp.zeros((B, H), jnp.float32)
    c = h
    for t in range(T):
        pre = gx_ref[t] + jnp.dot(
            h.astype(bf), whh1, preferred_element_type=jnp.float32
        )
        h, c = cell(pre, c)

    out_ref[...] = (
        jnp.dot(h.astype(bf), wfc_ref[...].astype(bf),
                preferred_element_type=jnp.float32)
        + bfc_ref[...]
    )


@jax.jit
def _forward(x, w_ih_0, w_hh_0, b_0, w_ih_1, w_hh_1, b_1, w_fc, b_fc):
    B, T, I = x.shape
    H = w_hh_0.shape[0]
    C = w_fc.shape[1]
    Bp = ((B + _CH - 1) // _CH) * _CH
    if Bp != B:
        x = jnp.pad(x, ((0, Bp - B), (0, 0), (0, 0)))

    body = functools.partial(_lstm_body, T=T, B=Bp, H=H)
    bcast = lambda shape: pl.BlockSpec(shape, lambda: (0,) * len(shape))
    out = pl.pallas_call(
        body,
        out_shape=jax.ShapeDtypeStruct((Bp, C), jnp.float32),
        grid=(),
        in_specs=[
            pl.BlockSpec(memory_space=pl.ANY),
            bcast((I, 4 * H)), bcast((H, 4 * H)), bcast((1, 4 * H)),
            bcast((H, 4 * H)), bcast((H, 4 * H)), bcast((1, 4 * H)),
            bcast((H, C)), bcast((1, C)),
        ],
        out_specs=bcast((Bp, C)),
        scratch_shapes=[
            pltpu.VMEM((T, Bp, 4 * H), jnp.float32),
            pltpu.VMEM((T, Bp, H), jnp.bfloat16),
            pltpu.VMEM((T, Bp, I), jnp.bfloat16),
            pltpu.VMEM((_DEPTH, _CH, T, I), jnp.float32),
            pltpu.SemaphoreType.DMA((_DEPTH,)),
        ],
    )(x, w_ih_0, w_hh_0, b_0, w_ih_1, w_hh_1, b_1, w_fc, b_fc)
    return out[:B]


def kernel(x, w_ih_0, w_hh_0, b_0, w_ih_1, w_hh_1, b_1, w_fc, b_fc):
    return _forward(x, w_ih_0, w_hh_0, b_0, w_ih_1, w_hh_1, b_1, w_fc, b_fc)


# speedup vs baseline: 1.4674x; 1.0012x over previous
import functools

import jax
import jax.numpy as jnp
from jax.experimental import pallas as pl
from jax.experimental.pallas import tpu as pltpu

_CH = 64
_DEPTH = 3


def _lstm_body(x_hbm, wih0_ref, whh0_ref, b0_ref, wih1_ref, whh1_ref, b1_ref,
               wfc_ref, bfc_ref, out_ref, gx_ref, seq_ref, xtm_ref,
               xchunk_ref, sem, *, T, B, H):
    bf = jnp.bfloat16
    nch = B // _CH
    col = jax.lax.broadcasted_iota(jnp.int32, (1, 4 * H), 1)
    scl = jnp.where((col >= 2 * H) & (col < 3 * H), 1.0, 0.5)

    def chunk_copy(ch):
        return pltpu.make_async_copy(
            x_hbm.at[pl.ds(ch * _CH, _CH)], xchunk_ref.at[ch % _DEPTH],
            sem.at[ch % _DEPTH],
        )

    for ch in range(min(_DEPTH, nch)):
        chunk_copy(ch).start()

    wih0 = (wih0_ref[...] * scl).astype(bf)
    b0 = b0_ref[...] * scl

    for ch in range(nch):
        slot = ch % _DEPTH
        chunk_copy(ch).wait()
        xt = jnp.swapaxes(xchunk_ref[slot], 0, 1).astype(bf)
        xtm_ref[:, pl.ds(ch * _CH, _CH), :] = xt
        gx = jnp.dot(xt.reshape(T * _CH, x_hbm.shape[-1]), wih0,
                     preferred_element_type=jnp.float32) + b0
        gx_ref[:, pl.ds(ch * _CH, _CH), :] = gx.reshape(T, _CH, 4 * H)
        if ch + _DEPTH < nch:
            chunk_copy(ch + _DEPTH).start()

    def cell(pre, c):
        tt = jnp.tanh(pre)
        i_g = 0.5 + 0.5 * tt[:, 0 * H:1 * H]
        f_g = 0.5 + 0.5 * tt[:, 1 * H:2 * H]
        g_g = tt[:, 2 * H:3 * H]
        o_g = 0.5 + 0.5 * tt[:, 3 * H:4 * H]
        c = f_g * c + i_g * g_g
        h = o_g * jnp.tanh(c)
        return h, c

    whh0 = (whh0_ref[...] * scl).astype(bf)
    h = jnp.zeros((B, H), jnp.float32)
    c = h
    for t in range(T):
        pre = gx_ref[t] + jnp.dot(
            h.astype(bf), whh0, preferred_element_type=jnp.float32
        )
        h, c = cell(pre, c)
        seq_ref[t] = h.astype(bf)

    wih1 = (wih1_ref[...] * scl).astype(bf)
    gx_ref[...] = (
        jnp.dot(seq_ref[...].reshape(T * B, H), wih1,
                preferred_element_type=jnp.float32).reshape(T, B, 4 * H)
        + b1_ref[...] * scl
    )

    whh1 = (whh1_ref[...] * scl).astype(bf)
    h = jnp.zeros((B, H), jnp.float32)
    c = h
    for t in range(T):
        pre = gx_ref[t] + jnp.dot(
            h.astype(bf), whh1, preferred_element_type=jnp.float32
        )
        h, c = cell(pre, c)

    out_ref[...] = (
        jnp.dot(h.astype(bf), wfc_ref[...].astype(bf),
                preferred_element_type=jnp.float32)
        + bfc_ref[...]
    )


@jax.jit
def _forward(x, w_ih_0, w_hh_0, b_0, w_ih_1, w_hh_1, b_1, w_fc, b_fc):
    B, T, I = x.shape
    H = w_hh_0.shape[0]
    C = w_fc.shape[1]
    Bp = ((B + _CH - 1) // _CH) * _CH
    if Bp != B:
        x = jnp.pad(x, ((0, Bp - B), (0, 0), (0, 0)))

    body = functools.partial(_lstm_body, T=T, B=Bp, H=H)
    bcast = lambda shape: pl.BlockSpec(shape, lambda: (0,) * len(shape))
    out = pl.pallas_call(
        body,
        out_shape=jax.ShapeDtypeStruct((Bp, C), jnp.float32),
        grid=(),
        in_specs=[
            pl.BlockSpec(memory_space=pl.ANY),
            bcast((I, 4 * H)), bcast((H, 4 * H)), bcast((1, 4 * H)),
            bcast((H, 4 * H)), bcast((H, 4 * H)), bcast((1, 4 * H)),
            bcast((H, C)), bcast((1, C)),
        ],
        out_specs=bcast((Bp, C)),
        scratch_shapes=[
            pltpu.VMEM((T, Bp, 4 * H), jnp.float32),
            pltpu.VMEM((T, Bp, H), jnp.bfloat16),
            pltpu.VMEM((T, Bp, I), jnp.bfloat16),
            pltpu.VMEM((_DEPTH, _CH, T, I), jnp.float32),
            pltpu.SemaphoreType.DMA((_DEPTH,)),
        ],
    )(x, w_ih_0, w_hh_0, b_0, w_ih_1, w_hh_1, b_1, w_fc, b_fc)
    return out[:B]


def kernel(x, w_ih_0, w_hh_0, b_0, w_ih_1, w_hh_1, b_1, w_fc, b_fc):
    return _forward(x, w_ih_0, w_hh_0, b_0, w_ih_1, w_hh_1, b_1, w_fc, b_fc)
